# baseline (speedup 1.0000x reference)
import functools

import jax
import jax.numpy as jnp
from jax import lax
from jax.experimental import pallas as pl
from jax.experimental.pallas import tpu as pltpu
from jax.experimental.pallas import tpu_sc as plsc

NC = 2
NS = 16
NW = NC * NS
L = 16
CHUNK = 128


def _emb_body(ids_hbm, table_hbm, out_hbm, idx_v, rows_v, sem, *, b_per_w, V, D):
  wid = lax.axis_index("s") * NC + lax.axis_index("c")
  base = wid * b_per_w
  n_chunks = b_per_w // CHUNK

  pltpu.sync_copy(ids_hbm.at[wid], idx_v)

  for j in range(n_chunks):
    for i in range(CHUNK // L):
      v = idx_v[j, pl.ds(i * L, L)]
      idx_v[j, pl.ds(i * L, L)] = jnp.where(v == -1, V - 1, v)

  copies = [
      pltpu.async_copy(
          table_hbm.at[idx_v.at[j]],
          rows_v.at[pl.ds(j * CHUNK, CHUNK)],
          sem,
      )
      for j in range(n_chunks)
  ]
  for c in copies:
    c.wait()

  pltpu.sync_copy(rows_v, out_hbm.at[pl.ds(base, b_per_w)])


@jax.jit
def kernel(class_ids, table):
  B = class_ids.shape[0]
  V, D = table.shape
  b_per_w = B // NW
  n_chunks = b_per_w // CHUNK

  ids3 = class_ids.astype(jnp.int32).reshape(NW, n_chunks, CHUNK)

  body = functools.partial(_emb_body, b_per_w=b_per_w, V=V, D=D)
  k = pl.kernel(
      body,
      out_type=jax.ShapeDtypeStruct((B, D), jnp.float32),
      mesh=plsc.VectorSubcoreMesh(core_axis_name="c", subcore_axis_name="s"),
      compiler_params=pltpu.CompilerParams(use_tc_tiling_on_sc=False),
      scratch_types=[
          pltpu.VMEM((n_chunks, CHUNK), jnp.int32),
          pltpu.VMEM((b_per_w, D), jnp.float32),
          pltpu.SemaphoreType.DMA,
      ],
  )
  return k(ids3, table)


# speedup vs baseline: 3.5905x; 3.5905x over previous
import functools

import jax
import jax.numpy as jnp
from jax import lax
from jax.experimental import pallas as pl
from jax.experimental.pallas import tpu as pltpu
from jax.experimental.pallas import tpu_sc as plsc

NC = 2
NS = 16
NW = NC * NS
WAVE = 16


def _emb_body(ids_hbm, tab_t_hbm, out_t_hbm, idx_v, cols_v, sem, *slots,
              b_per_w, V):
  wid = lax.axis_index("s") * NC + lax.axis_index("c")
  base = pl.multiple_of(wid * b_per_w, 128)

  pltpu.sync_copy(ids_hbm.at[wid], idx_v)
  rows16 = lax.iota(jnp.int32, 16)

  def wave(p):
    q = p // 4
    r = p % 4
    raw = idx_v[q, pl.ds(r * WAVE, WAVE)]
    ids16 = jnp.where(raw == -1, V - 1, raw)
    tcol16 = (ids16 // 128) * 128
    lane16 = lax.rem(ids16, 128)
    copies = []
    for j in range(WAVE):
      copies.append(
          pltpu.async_copy(
              tab_t_hbm.at[:, pl.ds(pl.multiple_of(tcol16[j], 128), 128)], slots[j], sem))
    for c in copies:
      c.wait()
    for j in range(WAVE):
      lane = jnp.full((16,), lane16[j], jnp.int32)
      col = jnp.full((16,), p * WAVE + j, jnp.int32)
      lo = plsc.load_gather(slots[j], [rows16, lane])
      hi = plsc.load_gather(slots[j], [rows16 + 16, lane])
      plsc.store_scatter(cols_v, [rows16, col], lo)
      plsc.store_scatter(cols_v, [rows16 + 16, col], hi)

  pl.loop(0, b_per_w // WAVE)(wave)

  pltpu.sync_copy(cols_v, out_t_hbm.at[:, pl.ds(base, b_per_w)])


@jax.jit
def kernel(class_ids, table):
  B = class_ids.shape[0]
  V, D = table.shape
  b_per_w = B // NW

  ids3 = class_ids.astype(jnp.int32).reshape(NW, 8, b_per_w // 8)
  tab_t = table.T

  body = functools.partial(_emb_body, b_per_w=b_per_w, V=V)
  k = pl.kernel(
      body,
      out_type=jax.ShapeDtypeStruct((D, B), jnp.float32),
      mesh=plsc.VectorSubcoreMesh(core_axis_name="c", subcore_axis_name="s"),
      compiler_params=pltpu.CompilerParams(needs_layout_passes=False),
      scratch_types=[
          pltpu.VMEM((8, b_per_w // 8), jnp.int32),
          pltpu.VMEM((D, b_per_w), jnp.float32),
          pltpu.SemaphoreType.DMA,
      ] + [pltpu.VMEM((D, 128), jnp.float32) for _ in range(WAVE)],
  )
  out_t = k(ids3, tab_t)
  return out_t.T
